# baseline (speedup 1.0000x reference)
import functools

import jax
import jax.numpy as jnp
from jax import lax
from jax.experimental import pallas as pl
from jax.experimental.pallas import tpu as pltpu
from jax.experimental.pallas import tpu_sc as plsc

_N = 10000
_E = 320000
_D = 128
_G = 64
_EP = 327680
_PADROW = 10200
_NW = 32
_CA = 64
_TA = _EP // (_CA * _NW)
_CB = 40
_TB = _EP // (_CB * _NW)
_CC = 128
_TN = _EP // (_CC * _NW)
_NP = 10240
_RPS = _NP // 16

_mesh = plsc.VectorSubcoreMesh(core_axis_name="c", subcore_axis_name="s")


def _fill(buf, rows, val16):
    @pl.loop(0, rows)
    def _(i):
        for r in range(8):
            buf[i, pl.ds(r * 16, 16)] = val16


def _zero_acc(zsrc, acc, sid, ncopy, zrows):
    @pl.loop(0, ncopy)
    def _(j):
        pltpu.sync_copy(zsrc, acc.at[pl.ds(sid * _RPS + j * zrows, zrows)])


@jax.jit
def _sc_cnt(colc):
    out = jax.ShapeDtypeStruct((2, _NP, _D), jnp.float32)
    scratch = (
        [pltpu.VMEM_SHARED((_NP, _D), jnp.float32)]
        + [pltpu.VMEM((_CC,), jnp.int32) for _ in range(4)]
        + [pltpu.VMEM((_CC, _D), jnp.float32)]
        + [pltpu.SemaphoreType.DMA for _ in range(4)]
    )

    def body(col_h, cnt_h, cacc, *r):
        cidx, ones = r[0:4], r[4]
        isem = r[5:9]
        cid = lax.axis_index("c")
        sid = lax.axis_index("s")
        wid = sid * 2 + cid
        base0 = wid

        _fill(ones, _CC, jnp.zeros((16,), jnp.float32))
        _zero_acc(ones, cacc, sid, _RPS // _CC, _CC)
        _fill(ones, _CC, jnp.ones((16,), jnp.float32))
        plsc.subcore_barrier()

        def idx_cp(k, s):
            return pltpu.make_async_copy(
                col_h.at[pl.ds((base0 + k * _NW) * _CC, _CC)], cidx[s], isem[s])

        def step(k, s, nxt2):
            if nxt2:
                idx_cp(k + 2, (s + 2) % 4).start()
            idx_cp(k, s).wait()
            pltpu.sync_copy(ones, cacc.at[cidx[s]], add=True)

        idx_cp(0, 0).start()
        idx_cp(1, 1).start()

        @pl.loop(0, _TN // 4 - 1)
        def _(j):
            k0 = 4 * j
            for i in range(4):
                step(k0 + i, i, True)

        for i in range(4):
            k = _TN - 4 + i
            step(k, i, k + 2 < _TN)

        plsc.subcore_barrier()
        pltpu.sync_copy(cacc.at[pl.ds(sid * _RPS, _RPS)],
                        cnt_h.at[cid, pl.ds(sid * _RPS, _RPS)])

    k = pl.kernel(body, mesh=_mesh, out_type=out, scratch_types=scratch)
    return k(colc)


@jax.jit
def _sc_phase_a(x, rowp, colp, earep):
    out = jax.ShapeDtypeStruct((2, _NP, _D), jnp.float32)
    scratch = (
        [pltpu.VMEM_SHARED((_NP, _D), jnp.float32)]
        + [pltpu.VMEM((_CA,), jnp.int32) for _ in range(8)]
        + [pltpu.VMEM((_CA, 16), jnp.float32) for _ in range(2)]
        + [pltpu.VMEM((_CA, _D), jnp.float32) for _ in range(2)]
        + [pltpu.SemaphoreType.DMA for _ in range(10)]
    )

    def body(x_h, row_h, col_h, ea_h, out_h, acc, *r):
        ridx, cidx = r[0:4], r[4:8]
        eab, gb = r[8:10], r[10:12]
        isem, easem, gsem, ssem = r[12:16], r[16:18], r[18:20], r[20:22]
        cid = lax.axis_index("c")
        sid = lax.axis_index("s")
        wid = sid * 2 + cid
        base0 = wid

        _fill(gb[0], _CA, jnp.zeros((16,), jnp.float32))
        _zero_acc(gb[0], acc, sid, _RPS // _CA, _CA)
        plsc.subcore_barrier()

        def idx_descs(k, s):
            b = (base0 + k * _NW) * _CA
            return [
                pltpu.make_async_copy(row_h.at[pl.ds(b, _CA)], ridx[s], isem[s]),
                pltpu.make_async_copy(col_h.at[pl.ds(b, _CA)], cidx[s], isem[s]),
            ]

        def ea_desc(k, e):
            b = (base0 + k * _NW) * _CA
            return pltpu.make_async_copy(ea_h.at[pl.ds(b, _CA)], eab[e], easem[e])

        def idx_start(k, s):
            for d in idx_descs(k, s):
                d.start()

        def idx_wait(k, s):
            for d in idx_descs(k, s):
                d.wait()

        def sc_desc(g, s):
            return pltpu.make_async_copy(gb[g], acc.at[cidx[s]], ssem[g])

        def step(k, s, g, first, nxt1, nxt2):
            if nxt2:
                idx_start(k + 2, (s + 2) % 4)
            if nxt1:
                idx_wait(k + 1, (s + 1) % 4)
                ea_desc(k + 1, 1 - g).wait()
                if not first:
                    sc_desc(1 - g, (s + 3) % 4).wait()
                pltpu.async_copy(x_h.at[ridx[(s + 1) % 4]], gb[1 - g],
                                 gsem[1 - g])
            pltpu.make_async_copy(x_h.at[ridx[s]], gb[g], gsem[g]).wait()

            @pl.loop(0, _CA)
            def _(c):
                eav = eab[g][c, pl.ds(0, 16)]
                for rr in range(8):
                    gb[g][c, pl.ds(rr * 16, 16)] = (
                        gb[g][c, pl.ds(rr * 16, 16)] * eav)

            pltpu.async_copy(gb[g], acc.at[cidx[s]], ssem[g], add=True)
            if nxt2:
                ea_desc(k + 2, g).start()

        idx_start(0, 0)
        idx_start(1, 1)
        ea_desc(0, 0).start()
        ea_desc(1, 1).start()
        idx_wait(0, 0)
        ea_desc(0, 0).wait()
        pltpu.async_copy(x_h.at[ridx[0]], gb[0], gsem[0])

        for i in range(4):
            step(i, i, i % 2, i == 0, True, True)

        @pl.loop(1, (_TA - 4) // 4)
        def _(j):
            k0 = 4 * j
            for i in range(4):
                step(k0 + i, i, i % 2, False, True, True)

        for i in range(4):
            k = _TA - 4 + i
            step(k, i, i % 2, False, k + 1 < _TA, k + 2 < _TA)
        sc_desc((_TA - 2) % 2, (_TA - 2) % 4).wait()
        sc_desc((_TA - 1) % 2, (_TA - 1) % 4).wait()

        plsc.subcore_barrier()
        pltpu.sync_copy(acc.at[pl.ds(sid * _RPS, _RPS)],
                        out_h.at[cid, pl.ds(sid * _RPS, _RPS)])

    k = pl.kernel(body, mesh=_mesh, out_type=out, scratch_types=scratch)
    return k(x, rowp, colp, earep)


@jax.jit
def _sc_phase_b(th, rowp, colp, earep):
    out = jax.ShapeDtypeStruct((2, _NP, _D), jnp.float32)
    scratch = (
        [pltpu.VMEM_SHARED((_NP, _D), jnp.float32)]
        + [pltpu.VMEM((_CB,), jnp.int32) for _ in range(8)]
        + [pltpu.VMEM((_CB, 16), jnp.float32) for _ in range(2)]
        + [pltpu.VMEM((_CB, _D), jnp.float32) for _ in range(4)]
        + [pltpu.SemaphoreType.DMA for _ in range(12)]
    )

    def body(th_h, row_h, col_h, ea_h, out_h, acc, *r):
        ridx, cidx, eab = r[0:4], r[4:8], r[8:10]
        ga, gbf = r[10:12], r[12:14]
        isem, easem = r[14:18], r[18:20]
        asem, bsem, ssem = r[20:22], r[22:24], r[24:26]
        cid = lax.axis_index("c")
        sid = lax.axis_index("s")
        wid = sid * 2 + cid
        base0 = wid

        _fill(ga[0], _CB, jnp.zeros((16,), jnp.float32))
        _zero_acc(ga[0], acc, sid, _RPS // _CB, _CB)
        plsc.subcore_barrier()

        def idx_descs(k, s):
            b = (base0 + k * _NW) * _CB
            return [
                pltpu.make_async_copy(row_h.at[pl.ds(b, _CB)], ridx[s], isem[s]),
                pltpu.make_async_copy(col_h.at[pl.ds(b, _CB)], cidx[s], isem[s]),
            ]

        def ea_desc(k, e):
            b = (base0 + k * _NW) * _CB
            return pltpu.make_async_copy(ea_h.at[pl.ds(b, _CB)], eab[e], easem[e])

        def idx_start(k, s):
            for d in idx_descs(k, s):
                d.start()

        def idx_wait(k, s):
            for d in idx_descs(k, s):
                d.wait()

        def g_start(s, g):
            pltpu.async_copy(th_h.at[ridx[s]], ga[g], asem[g])
            pltpu.async_copy(th_h.at[cidx[s]], gbf[g], bsem[g])

        def g_wait(s, g):
            pltpu.make_async_copy(th_h.at[ridx[s]], ga[g], asem[g]).wait()
            pltpu.make_async_copy(th_h.at[cidx[s]], gbf[g], bsem[g]).wait()

        def sc_desc(g, s):
            return pltpu.make_async_copy(ga[g], acc.at[ridx[s]], ssem[g])

        def step(k, s, g, first, nxt1, nxt2):
            if nxt2:
                idx_start(k + 2, (s + 2) % 4)
            if nxt1:
                idx_wait(k + 1, (s + 1) % 4)
                ea_desc(k + 1, 1 - g).wait()
                if not first:
                    sc_desc(1 - g, (s + 3) % 4).wait()
                g_start((s + 1) % 4, 1 - g)
            g_wait(s, g)

            @pl.loop(0, _CB)
            def _(c):
                eav = eab[g][c, pl.ds(0, 16)]
                for rr in range(8):
                    a = ga[g][c, pl.ds(rr * 16, 16)]
                    b_ = gbf[g][c, pl.ds(rr * 16, 16)]
                    ga[g][c, pl.ds(rr * 16, 16)] = jnp.abs(a - b_) * eav

            pltpu.async_copy(ga[g], acc.at[ridx[s]], ssem[g], add=True)
            if nxt2:
                ea_desc(k + 2, g).start()

        idx_start(0, 0)
        idx_start(1, 1)
        ea_desc(0, 0).start()
        ea_desc(1, 1).start()
        idx_wait(0, 0)
        ea_desc(0, 0).wait()
        g_start(0, 0)

        for i in range(4):
            step(i, i, i % 2, i == 0, True, True)

        @pl.loop(1, (_TB - 4) // 4)
        def _(j):
            k0 = 4 * j
            for i in range(4):
                step(k0 + i, i, i % 2, False, True, True)

        for i in range(4):
            k = _TB - 4 + i
            step(k, i, i % 2, False, k + 1 < _TB, k + 2 < _TB)
        sc_desc((_TB - 2) % 2, (_TB - 2) % 4).wait()
        sc_desc((_TB - 1) % 2, (_TB - 1) % 4).wait()

        plsc.subcore_barrier()
        pltpu.sync_copy(acc.at[pl.ds(sid * _RPS, _RPS)],
                        out_h.at[cid, pl.ds(sid * _RPS, _RPS)])

    k = pl.kernel(body, mesh=_mesh, out_type=out, scratch_types=scratch)
    return k(th, rowp, colp, earep)


_NB = 25
_BR = _N // _NB


def _tau0_body(x_ref, gw_ref, gb_ref, th_ref):
    th_ref[...] = jax.nn.sigmoid(
        jnp.dot(x_ref[...], gw_ref[...].T, preferred_element_type=jnp.float32)
        + gb_ref[...][None, :])


def _tau0(x, gw, gb):
    return pl.pallas_call(
        _tau0_body,
        grid=(_NB,),
        in_specs=[
            pl.BlockSpec((_BR, _D), lambda i: (i, 0)),
            pl.BlockSpec((_D, _D), lambda i: (0, 0)),
            pl.BlockSpec((_D,), lambda i: (0,)),
        ],
        out_specs=pl.BlockSpec((_BR, _D), lambda i: (i, 0)),
        out_shape=jax.ShapeDtypeStruct((_N, _D), jnp.float32),
    )(x, gw, gb)


def _combine_body(with_th, x_ref, pa_ref, pb_ref, cnt_ref, lw_ref, lb_ref,
                  gw_ref, gb_ref, xo_ref, *maybe_th):
    x = x_ref[...]
    pa = pa_ref[...]
    pb = pb_ref[...]
    cnt = cnt_ref[...]
    summed = pa[0] + pa[1]
    c = cnt[0][:, 0:1] + cnt[1][:, 0:1]
    aggr = summed / jnp.clip(c, 1.0)
    out = jnp.maximum(
        jnp.dot(aggr, lw_ref[...].T, preferred_element_type=jnp.float32)
        + lb_ref[...][None, :], 0.0)
    tau = jnp.tanh(pb[0] + pb[1])
    xn = (1.0 - tau) * x + tau * out + x
    xo_ref[...] = xn
    if with_th:
        maybe_th[0][...] = jax.nn.sigmoid(
            jnp.dot(xn, gw_ref[...].T, preferred_element_type=jnp.float32)
            + gb_ref[...][None, :])


def _combine(x, pa, pb, cnt, lw, lb, gw, gb, with_th):
    outs = [jax.ShapeDtypeStruct((_N, _D), jnp.float32)]
    out_specs = [pl.BlockSpec((_BR, _D), lambda i: (i, 0))]
    if with_th:
        outs.append(jax.ShapeDtypeStruct((_N, _D), jnp.float32))
        out_specs.append(pl.BlockSpec((_BR, _D), lambda i: (i, 0)))
    res = pl.pallas_call(
        functools.partial(_combine_body, with_th),
        grid=(_NB,),
        in_specs=[
            pl.BlockSpec((_BR, _D), lambda i: (i, 0)),
            pl.BlockSpec((2, _BR, _D), lambda i: (0, i, 0)),
            pl.BlockSpec((2, _BR, _D), lambda i: (0, i, 0)),
            pl.BlockSpec((2, _BR, _D), lambda i: (0, i, 0)),
            pl.BlockSpec((_D, _D), lambda i: (0, 0)),
            pl.BlockSpec((_D,), lambda i: (0,)),
            pl.BlockSpec((_D, _D), lambda i: (0, 0)),
            pl.BlockSpec((_D,), lambda i: (0,)),
        ],
        out_specs=out_specs,
        out_shape=outs,
    )(x, pa, pb, cnt, lw, lb, gw, gb)
    return res if with_th else (res[0], None)


def _readout_body(bids_ref, x_ref, cw_ref, cb_ref, o_ref, acc_ref, cnt_ref):
    i = pl.program_id(0)

    @pl.when(i == 0)
    def _():
        acc_ref[...] = jnp.zeros_like(acc_ref)
        cnt_ref[...] = jnp.zeros_like(cnt_ref)

    bids = bids_ref[0, 0, :]
    rows = bids.shape[0]
    gi = jax.lax.broadcasted_iota(jnp.int32, (_G, rows), 0)
    onehot = (gi == bids[None, :]).astype(jnp.float32)
    acc_ref[...] += jnp.dot(onehot, x_ref[...], preferred_element_type=jnp.float32)
    cnt_ref[...] += jnp.sum(onehot, axis=1, keepdims=True)

    @pl.when(i == pl.num_programs(0) - 1)
    def _():
        emb = acc_ref[...] / jnp.clip(cnt_ref[...], 1.0)
        o_ref[...] = jnp.dot(emb, cw_ref[...].T,
                             preferred_element_type=jnp.float32) + cb_ref[...][None, :]


def _readout(x, batch, cls_w, cls_b):
    bids = batch.astype(jnp.int32).reshape(_NB, 1, _BR)
    return pl.pallas_call(
        _readout_body,
        grid=(_NB,),
        in_specs=[
            pl.BlockSpec((1, 1, _BR), lambda i: (i, 0, 0)),
            pl.BlockSpec((_BR, _D), lambda i: (i, 0)),
            pl.BlockSpec((6, _D), lambda i: (0, 0)),
            pl.BlockSpec((6,), lambda i: (0,)),
        ],
        out_specs=pl.BlockSpec((_G, 6), lambda i: (0, 0)),
        out_shape=jax.ShapeDtypeStruct((_G, 6), jnp.float32),
        scratch_shapes=[
            pltpu.VMEM((_G, _D), jnp.float32),
            pltpu.VMEM((_G, 1), jnp.float32),
        ],
    )(bids, x, cls_w, cls_b)


def kernel(x, edge_index, edge_attr, batch,
           lin_w0, lin_b0, gate_w0, gate_b0,
           lin_w1, lin_b1, gate_w1, gate_b1,
           lin_w2, lin_b2, gate_w2, gate_b2,
           cls_w, cls_b):
    row = edge_index[0].astype(jnp.int32)
    col = edge_index[1].astype(jnp.int32)
    pad = _EP - _E
    zpad = jnp.zeros((pad,), jnp.int32)
    rowp = jnp.concatenate([row, zpad])
    colp = jnp.concatenate([col, zpad])
    colc = jnp.concatenate([col, jnp.full((pad,), _PADROW, jnp.int32)])
    earep = jnp.concatenate(
        [jnp.broadcast_to(edge_attr.astype(jnp.float32), (_E, 16)),
         jnp.zeros((pad, 16), jnp.float32)])
    params = [
        (lin_w0, lin_b0, gate_w0, gate_b0),
        (lin_w1, lin_b1, gate_w1, gate_b1),
        (lin_w2, lin_b2, gate_w2, gate_b2),
    ]
    th = _tau0(x, gate_w0, gate_b0)
    cnt = _sc_cnt(colc)
    for i, (lw, lb, gw, gb) in enumerate(params):
        pa = _sc_phase_a(x, rowp, colp, earep)
        pb = _sc_phase_b(th, rowp, colp, earep)
        with_th = i < 2
        ngw, ngb = (params[i + 1][2], params[i + 1][3]) if with_th else (gw, gb)
        x, th = _combine(x, pa, pb, cnt, lw, lb, ngw, ngb, with_th)
    return _readout(x, batch, cls_w, cls_b)


# speedup vs baseline: 1.3284x; 1.3284x over previous
import functools

import jax
import jax.numpy as jnp
from jax import lax
from jax.experimental import pallas as pl
from jax.experimental.pallas import tpu as pltpu
from jax.experimental.pallas import tpu_sc as plsc

_N = 10000
_E = 320000
_D = 128
_G = 64
_C = 128
_NCHUNK = _E // _C
_NW = 32
_NP = 10240
_RPS = _NP // 16
_CB = 64
_NCHUNK_B = _E // _CB

_mesh = plsc.VectorSubcoreMesh(core_axis_name="c", subcore_axis_name="s")


def _zero16():
    return jnp.zeros((16,), jnp.float32)


def _chunk_loop(wid, body, nchunk=_NCHUNK):
    nfull = nchunk // _NW
    rem = nchunk - nfull * _NW
    trips = nfull + jnp.where(wid < rem, 1, 0)

    def fbody(k, carry):
        body(wid + k * _NW)
        return carry

    lax.fori_loop(0, trips, fbody, 0)


def _zero_shared(zsrc, acc, sid, ncopy, zrows):
    @pl.loop(0, ncopy)
    def _(j):
        pltpu.sync_copy(zsrc, acc.at[pl.ds(sid * _RPS + j * zrows, zrows)])


@jax.jit
def _sc_cnt(col):
    out = jax.ShapeDtypeStruct((2, _NP, _D), jnp.float32)
    scratch = [
        pltpu.VMEM_SHARED((_NP, _D), jnp.float32),
        pltpu.VMEM((_C,), jnp.int32),
        pltpu.VMEM((_C, _D), jnp.float32),
    ]

    def body(col_h, cnt_h, cacc, cidx, ones):
        cid = lax.axis_index("c")
        sid = lax.axis_index("s")
        wid = sid * 2 + cid
        z16 = _zero16()
        o16 = jnp.ones((16,), jnp.float32)

        @pl.loop(0, _C)
        def _(i):
            for r in range(8):
                ones[i, pl.ds(r * 16, 16)] = z16

        _zero_shared(ones, cacc, sid, 5, _C)

        @pl.loop(0, _C)
        def _(i):
            for r in range(8):
                ones[i, pl.ds(r * 16, 16)] = o16

        plsc.subcore_barrier()

        def chunk(g):
            base = g * _C
            pltpu.sync_copy(col_h.at[pl.ds(base, _C)], cidx)
            pltpu.sync_copy(ones, cacc.at[cidx], add=True)

        _chunk_loop(wid, chunk)
        plsc.subcore_barrier()
        pltpu.sync_copy(cacc.at[pl.ds(sid * _RPS, _RPS)],
                        cnt_h.at[cid, pl.ds(sid * _RPS, _RPS)])

    k = pl.kernel(body, mesh=_mesh, out_type=out, scratch_types=scratch)
    return k(col)


@jax.jit
def _sc_phase_a(x, row, col, earep):
    out = jax.ShapeDtypeStruct((2, _NP, _D), jnp.float32)
    scratch = [
        pltpu.VMEM_SHARED((_NP, _D), jnp.float32),
        pltpu.VMEM((_C,), jnp.int32),
        pltpu.VMEM((_C,), jnp.int32),
        pltpu.VMEM((_C, 16), jnp.float32),
        pltpu.VMEM((_C, _D), jnp.float32),
        pltpu.SemaphoreType.DMA,
    ]

    def body(x_h, row_h, col_h, ea_h, out_h, acc, ridx, cidx, eab, gbuf, sem):
        cid = lax.axis_index("c")
        sid = lax.axis_index("s")
        wid = sid * 2 + cid
        z16 = _zero16()

        @pl.loop(0, _C)
        def _(i):
            for r in range(8):
                gbuf[i, pl.ds(r * 16, 16)] = z16

        _zero_shared(gbuf, acc, sid, 5, _C)
        plsc.subcore_barrier()

        def chunk(g):
            base = g * _C
            pltpu.sync_copy(row_h.at[pl.ds(base, _C)], ridx)
            pltpu.sync_copy(col_h.at[pl.ds(base, _C)], cidx)
            pltpu.sync_copy(ea_h.at[pl.ds(base, _C)], eab)
            pltpu.async_copy(x_h.at[ridx], gbuf, sem).wait()

            @pl.loop(0, _C)
            def _(c):
                eav = eab[c, pl.ds(0, 16)]
                for rr in range(8):
                    gbuf[c, pl.ds(rr * 16, 16)] = (
                        gbuf[c, pl.ds(rr * 16, 16)] * eav)

            pltpu.sync_copy(gbuf, acc.at[cidx], add=True)

        _chunk_loop(wid, chunk)
        plsc.subcore_barrier()
        pltpu.sync_copy(acc.at[pl.ds(sid * _RPS, _RPS)],
                        out_h.at[cid, pl.ds(sid * _RPS, _RPS)])

    k = pl.kernel(body, mesh=_mesh, out_type=out, scratch_types=scratch)
    return k(x, row, col, earep)


@jax.jit
def _sc_phase_b(th, row, col, earep):
    out = jax.ShapeDtypeStruct((2, _NP, _D), jnp.float32)
    scratch = [
        pltpu.VMEM_SHARED((_NP, _D), jnp.float32),
        pltpu.VMEM((_CB,), jnp.int32),
        pltpu.VMEM((_CB,), jnp.int32),
        pltpu.VMEM((_CB, 16), jnp.float32),
        pltpu.VMEM((_CB, _D), jnp.float32),
        pltpu.VMEM((_CB, _D), jnp.float32),
        pltpu.SemaphoreType.DMA,
        pltpu.SemaphoreType.DMA,
    ]

    def body(th_h, row_h, col_h, ea_h, out_h, acc, ridx, cidx, eab, ga, gb,
             sema, semb):
        cid = lax.axis_index("c")
        sid = lax.axis_index("s")
        wid = sid * 2 + cid
        z16 = _zero16()

        @pl.loop(0, _CB)
        def _(i):
            for r in range(8):
                ga[i, pl.ds(r * 16, 16)] = z16

        _zero_shared(ga, acc, sid, 10, _CB)
        plsc.subcore_barrier()

        def chunk(g):
            base = g * _CB
            pltpu.sync_copy(row_h.at[pl.ds(base, _CB)], ridx)
            pltpu.sync_copy(col_h.at[pl.ds(base, _CB)], cidx)
            pltpu.sync_copy(ea_h.at[pl.ds(base, _CB)], eab)
            ca = pltpu.async_copy(th_h.at[ridx], ga, sema)
            cb = pltpu.async_copy(th_h.at[cidx], gb, semb)
            ca.wait()
            cb.wait()

            @pl.loop(0, _CB)
            def _(c):
                eav = eab[c, pl.ds(0, 16)]
                for rr in range(8):
                    a = ga[c, pl.ds(rr * 16, 16)]
                    b_ = gb[c, pl.ds(rr * 16, 16)]
                    ga[c, pl.ds(rr * 16, 16)] = jnp.abs(a - b_) * eav

            pltpu.sync_copy(ga, acc.at[ridx], add=True)

        _chunk_loop(wid, chunk, _NCHUNK_B)
        plsc.subcore_barrier()
        pltpu.sync_copy(acc.at[pl.ds(sid * _RPS, _RPS)],
                        out_h.at[cid, pl.ds(sid * _RPS, _RPS)])

    k = pl.kernel(body, mesh=_mesh, out_type=out, scratch_types=scratch)
    return k(th, row, col, earep)


_NB = 25
_BR = _N // _NB


def _tau0_body(x_ref, gw_ref, gb_ref, th_ref):
    th_ref[...] = jax.nn.sigmoid(
        jnp.dot(x_ref[...], gw_ref[...].T, preferred_element_type=jnp.float32)
        + gb_ref[...][None, :])


def _tau0(x, gw, gb):
    return pl.pallas_call(
        _tau0_body,
        grid=(_NB,),
        in_specs=[
            pl.BlockSpec((_BR, _D), lambda i: (i, 0)),
            pl.BlockSpec((_D, _D), lambda i: (0, 0)),
            pl.BlockSpec((_D,), lambda i: (0,)),
        ],
        out_specs=pl.BlockSpec((_BR, _D), lambda i: (i, 0)),
        out_shape=jax.ShapeDtypeStruct((_N, _D), jnp.float32),
    )(x, gw, gb)


def _combine_body(with_th, x_ref, pa_ref, pb_ref, cnt_ref, lw_ref, lb_ref,
                  gw_ref, gb_ref, xo_ref, *maybe_th):
    x = x_ref[...]
    pa = pa_ref[...]
    pb = pb_ref[...]
    cnt = cnt_ref[...]
    summed = pa[0] + pa[1]
    c = cnt[0][:, 0:1] + cnt[1][:, 0:1]
    aggr = summed / jnp.clip(c, 1.0)
    out = jnp.maximum(
        jnp.dot(aggr, lw_ref[...].T, preferred_element_type=jnp.float32)
        + lb_ref[...][None, :], 0.0)
    tau = jnp.tanh(pb[0] + pb[1])
    xn = (1.0 - tau) * x + tau * out + x
    xo_ref[...] = xn
    if with_th:
        maybe_th[0][...] = jax.nn.sigmoid(
            jnp.dot(xn, gw_ref[...].T, preferred_element_type=jnp.float32)
            + gb_ref[...][None, :])


def _combine(x, pa, pb, cnt, lw, lb, gw, gb, with_th):
    outs = [jax.ShapeDtypeStruct((_N, _D), jnp.float32)]
    out_specs = [pl.BlockSpec((_BR, _D), lambda i: (i, 0))]
    if with_th:
        outs.append(jax.ShapeDtypeStruct((_N, _D), jnp.float32))
        out_specs.append(pl.BlockSpec((_BR, _D), lambda i: (i, 0)))
    res = pl.pallas_call(
        functools.partial(_combine_body, with_th),
        grid=(_NB,),
        in_specs=[
            pl.BlockSpec((_BR, _D), lambda i: (i, 0)),
            pl.BlockSpec((2, _BR, _D), lambda i: (0, i, 0)),
            pl.BlockSpec((2, _BR, _D), lambda i: (0, i, 0)),
            pl.BlockSpec((2, _BR, _D), lambda i: (0, i, 0)),
            pl.BlockSpec((_D, _D), lambda i: (0, 0)),
            pl.BlockSpec((_D,), lambda i: (0,)),
            pl.BlockSpec((_D, _D), lambda i: (0, 0)),
            pl.BlockSpec((_D,), lambda i: (0,)),
        ],
        out_specs=out_specs,
        out_shape=outs,
    )(x, pa, pb, cnt, lw, lb, gw, gb)
    return res if with_th else (res[0], None)


def _readout_body(bids_ref, x_ref, cw_ref, cb_ref, o_ref, acc_ref, cnt_ref):
    i = pl.program_id(0)

    @pl.when(i == 0)
    def _():
        acc_ref[...] = jnp.zeros_like(acc_ref)
        cnt_ref[...] = jnp.zeros_like(cnt_ref)

    bids = bids_ref[0, 0, :]
    rows = bids.shape[0]
    gi = jax.lax.broadcasted_iota(jnp.int32, (_G, rows), 0)
    onehot = (gi == bids[None, :]).astype(jnp.float32)
    acc_ref[...] += jnp.dot(onehot, x_ref[...], preferred_element_type=jnp.float32)
    cnt_ref[...] += jnp.sum(onehot, axis=1, keepdims=True)

    @pl.when(i == pl.num_programs(0) - 1)
    def _():
        emb = acc_ref[...] / jnp.clip(cnt_ref[...], 1.0)
        o_ref[...] = jnp.dot(emb, cw_ref[...].T,
                             preferred_element_type=jnp.float32) + cb_ref[...][None, :]


def _readout(x, batch, cls_w, cls_b):
    bids = batch.astype(jnp.int32).reshape(_NB, 1, _BR)
    return pl.pallas_call(
        _readout_body,
        grid=(_NB,),
        in_specs=[
            pl.BlockSpec((1, 1, _BR), lambda i: (i, 0, 0)),
            pl.BlockSpec((_BR, _D), lambda i: (i, 0)),
            pl.BlockSpec((6, _D), lambda i: (0, 0)),
            pl.BlockSpec((6,), lambda i: (0,)),
        ],
        out_specs=pl.BlockSpec((_G, 6), lambda i: (0, 0)),
        out_shape=jax.ShapeDtypeStruct((_G, 6), jnp.float32),
        scratch_shapes=[
            pltpu.VMEM((_G, _D), jnp.float32),
            pltpu.VMEM((_G, 1), jnp.float32),
        ],
    )(bids, x, cls_w, cls_b)


def kernel(x, edge_index, edge_attr, batch,
           lin_w0, lin_b0, gate_w0, gate_b0,
           lin_w1, lin_b1, gate_w1, gate_b1,
           lin_w2, lin_b2, gate_w2, gate_b2,
           cls_w, cls_b):
    row = edge_index[0].astype(jnp.int32)
    col = edge_index[1].astype(jnp.int32)
    earep = jnp.broadcast_to(edge_attr.astype(jnp.float32), (_E, 16))
    params = [
        (lin_w0, lin_b0, gate_w0, gate_b0),
        (lin_w1, lin_b1, gate_w1, gate_b1),
        (lin_w2, lin_b2, gate_w2, gate_b2),
    ]
    th = _tau0(x, gate_w0, gate_b0)
    cnt = _sc_cnt(col)
    for i, (lw, lb, gw, gb) in enumerate(params):
        pa = _sc_phase_a(x, row, col, earep)
        pb = _sc_phase_b(th, row, col, earep)
        with_th = i < 2
        ngw, ngb = (params[i + 1][2], params[i + 1][3]) if with_th else (gw, gb)
        x, th = _combine(x, pa, pb, cnt, lw, lb, ngw, ngb, with_th)
    return _readout(x, batch, cls_w, cls_b)
